# baseline (speedup 1.0000x reference)
import jax
import jax.numpy as jnp
from jax import lax
from jax.experimental import pallas as pl
from jax.experimental.pallas import tpu as pltpu
from jax.experimental.pallas import tpu_sc as plsc

B = 16384
F = 26
VOCAB = 100000
D = 32

NC = 2
NS = 16
NW = NC * NS
L = 16
CB = 128
NBC = B // CB
NCHUNK = F * NBC
CPW = NCHUNK // NW
TROWS = F * VOCAB // 4


def _body(idx_hbm, tab_hbm, out_hbm, idx_v, g2_v, rows_v, outb_v, sem):
    wid = lax.axis_index("s") * NC + lax.axis_index("c")
    lane = lax.iota(jnp.int32, L)
    ib128 = lane * CB

    def chunk(i, carry):
        c = wid * CPW + i
        f = c // NBC
        bc = c - f * NBC

        pltpu.sync_copy(idx_hbm.at[f, pl.ds(bc * CB, CB)], idx_v)

        offs = []
        for g in range(CB // L):
            raw = idx_v[pl.ds(g * L, L)]
            gid = f * VOCAB + jnp.clip(raw, 0, VOCAB - 1)
            g2_v[pl.ds(g * L, L)] = lax.shift_right_logical(gid, 2)
            offs.append(lax.shift_left(lax.bitwise_and(gid, 3), 5))

        pltpu.async_copy(tab_hbm.at[g2_v], rows_v, sem).wait()

        for l in range(CB):
            o = offs[l // L][l % L]
            for h in range(D // L):
                a = rows_v[l, pl.ds(o + h * L, L)]
                plsc.store_scatter(outb_v, [ib128 + (h * L * CB + l)], a)

        for dt in range(D // 8):
            pltpu.sync_copy(outb_v.at[pl.ds(dt * 1024, 1024)],
                            out_hbm.at[f, dt, bc])
        return carry

    lax.fori_loop(0, CPW, chunk, 0)


def kernel(indices, tables):
    idx_t = jnp.swapaxes(indices, 0, 1).astype(jnp.int32)
    tab128 = tables.reshape(TROWS, 128)
    mesh = plsc.VectorSubcoreMesh(
        core_axis_name="c", subcore_axis_name="s",
        num_cores=NC, num_subcores=NS,
    )
    f = pl.kernel(
        _body,
        out_type=jax.ShapeDtypeStruct((F, D // 8, NBC, 8 * CB), jnp.float32),
        mesh=mesh,
        scratch_types=[
            pltpu.VMEM((CB,), jnp.int32),
            pltpu.VMEM((CB,), jnp.int32),
            pltpu.VMEM((CB, 128), jnp.float32),
            pltpu.VMEM((D * CB,), jnp.float32),
            pltpu.SemaphoreType.DMA,
        ],
        compiler_params=pltpu.CompilerParams(
            use_tc_tiling_on_sc=False, needs_layout_passes=False),
    )
    out4d = f(idx_t, tab128)
    out = jnp.transpose(
        out4d.reshape(F, D // 8, NBC, 8, CB), (2, 4, 0, 1, 3)
    ).reshape(B, F, D)
    return out


# speedup vs baseline: 2.1119x; 2.1119x over previous
import jax
import jax.numpy as jnp
from jax import lax
from jax.experimental import pallas as pl
from jax.experimental.pallas import tpu as pltpu
from jax.experimental.pallas import tpu_sc as plsc

B = 16384
F = 26
VOCAB = 100000
D = 32

NC = 2
NS = 16
NW = NC * NS
L = 16
NP = F * D
PPT = NP // NW
BH = B // 2
NR = BH // 128
OROWS = F * D * B // 128 // 8 * 8 // 128 * 128
OROWS = B * F * D // 128


def _body(idx_hbm, tab_hbm, out_hbm, idx_v, strip_v, stg_v, row_v, sem):
    wid = lax.axis_index("s") * NC + lax.axis_index("c")
    lane = lax.iota(jnp.int32, L)

    def strip(i, carry):
        p = wid * PPT + i
        f = p // D
        d = p - f * D
        pltpu.sync_copy(tab_hbm.at[f, d], strip_v)
        base = f * 4096 + (d // 8) * 1024 + (d % 8)

        for h in range(2):
            pltpu.sync_copy(idx_hbm.at[f, pl.ds(h * BH, BH)], idx_v)

            def row(bc, carry2):
                for j in range(8):
                    raw = idx_v[pl.ds(bc * 128 + j * L, L)]
                    v = jnp.clip(raw, 0, VOCAB - 1)
                    stg_v[bc, pl.ds(j * L, L)] = plsc.load_gather(
                        strip_v, [v])
                return carry2
            lax.fori_loop(0, NR, row, 0)

            for g in range(NR // L):
                row_v[pl.ds(g * L, L)] = (
                    base + h * 512 + (g * L + lane) * 8)
            pltpu.async_copy(stg_v, out_hbm.at[row_v], sem).wait()
        return carry

    lax.fori_loop(0, PPT, strip, 0)


def kernel(indices, tables):
    idx_t = jnp.swapaxes(indices, 0, 1).astype(jnp.int32)
    tab_t = jnp.swapaxes(tables, 1, 2)
    mesh = plsc.VectorSubcoreMesh(
        core_axis_name="c", subcore_axis_name="s",
        num_cores=NC, num_subcores=NS,
    )
    fn = pl.kernel(
        _body,
        out_type=jax.ShapeDtypeStruct((OROWS, 128), jnp.float32),
        mesh=mesh,
        scratch_types=[
            pltpu.VMEM((BH,), jnp.int32),
            pltpu.VMEM((VOCAB,), jnp.float32),
            pltpu.VMEM((NR, 128), jnp.float32),
            pltpu.VMEM((NR,), jnp.int32),
            pltpu.SemaphoreType.DMA,
        ],
        compiler_params=pltpu.CompilerParams(
            use_tc_tiling_on_sc=False, needs_layout_passes=False),
    )
    out2d = fn(idx_t, tab_t)
    out = jnp.transpose(
        out2d.reshape(F, D // 8, B // 128, 8, 128), (2, 4, 0, 1, 3)
    ).reshape(B, F, D)
    return out
